# Initial kernel scaffold; baseline (speedup 1.0000x reference)
import jax
import jax.numpy as jnp
from jax.experimental import pallas as pl


def kernel(x, edge_index, params):
    raise NotImplementedError("write your pallas kernel here")


# speedup vs baseline: 2.7478x; 2.7478x over previous
import functools

import jax
import jax.numpy as jnp
from jax import lax
from jax.experimental import pallas as pl
from jax.experimental.pallas import tpu as pltpu
from jax.experimental.pallas import tpu_sc as plsc

N_NODES = 10000
N_EDGES = 160000
D_FEAT = 256
HIDDEN = 512

NC = 2
NT = 16
CW = 128

ACC_ROWS = 10240
ROWS_PER_TILE = ACC_ROWS // NT
LAST_ROWS = N_NODES - (NT - 1) * ROWS_PER_TILE
EDGE_BATCH = 128
E_PAD = 163840
NB = E_PAD // (NT * EDGE_BATCH)


@functools.lru_cache(maxsize=None)
def _sc_agg(n_chunks):
    cps = n_chunks // NC
    mesh = plsc.VectorSubcoreMesh(
        core_axis_name="c", subcore_axis_name="s", num_cores=NC, num_subcores=NT
    )
    out_type = [jax.ShapeDtypeStruct((N_NODES, CW), jnp.float32)
                for _ in range(n_chunks)]
    scratch = [
        pltpu.VMEM((NB, EDGE_BATCH), jnp.int32),
        pltpu.VMEM((NB, EDGE_BATCH), jnp.int32),
        pltpu.VMEM((EDGE_BATCH, CW), jnp.float32),
        pltpu.VMEM((16, CW), jnp.float32),
        pltpu.VMEM_SHARED((ACC_ROWS, CW), jnp.float32),
        pltpu.SemaphoreType.DMA,
    ]

    @functools.partial(pl.kernel, out_type=out_type, mesh=mesh,
                       scratch_types=scratch)
    def agg_kernel(*refs):
        h_refs = refs[:n_chunks]
        src_hbm = refs[n_chunks]
        dst_hbm = refs[n_chunks + 1]
        out_refs = refs[n_chunks + 2: 2 * n_chunks + 2]
        sidx, didx, gbuf, zbuf, acc, sem = refs[2 * n_chunks + 2:]

        cid = lax.axis_index("c")
        sid = lax.axis_index("s")
        base = sid * ROWS_PER_TILE

        pltpu.sync_copy(src_hbm.at[sid], sidx)
        pltpu.sync_copy(dst_hbm.at[sid], didx)

        for i in range(16):
            for j in range(CW // 16):
                zbuf[i, pl.ds(j * 16, 16)] = jnp.zeros((16,), jnp.float32)

        def process_chunk(h_ref, out_ref):
            def zero_body(k, carry):
                pltpu.sync_copy(zbuf, acc.at[pl.ds(base + k * 16, 16)])
                return carry
            lax.fori_loop(0, ROWS_PER_TILE // 16, zero_body, 0)
            plsc.subcore_barrier()

            def edge_body(b, carry):
                pltpu.async_copy(h_ref.at[sidx.at[b]], gbuf, sem).wait()
                pltpu.sync_copy(gbuf, acc.at[didx.at[b]], add=True)
                return carry
            lax.fori_loop(0, NB, edge_body, 0)
            plsc.subcore_barrier()

            @pl.when(sid < NT - 1)
            def _():
                pltpu.sync_copy(acc.at[pl.ds(base, ROWS_PER_TILE)],
                                out_ref.at[pl.ds(base, ROWS_PER_TILE)])

            @pl.when(sid == NT - 1)
            def _():
                pltpu.sync_copy(
                    acc.at[pl.ds((NT - 1) * ROWS_PER_TILE, LAST_ROWS)],
                    out_ref.at[pl.ds((NT - 1) * ROWS_PER_TILE, LAST_ROWS)])
            plsc.subcore_barrier()

        for j in range(cps):
            for side in range(NC):
                chunk = side * cps + j

                @pl.when(cid == side)
                def _(chunk=chunk):
                    process_chunk(h_refs[chunk], out_refs[chunk])

    return agg_kernel


_BR = 2000


@functools.lru_cache(maxsize=None)
def _mlp(n_in_chunks, final):
    d_in = n_in_chunks * CW
    n_out_chunks = HIDDEN // CW

    def body(*refs):
        scale_ref = refs[0]
        h_refs = refs[1:1 + n_in_chunks]
        a_refs = refs[1 + n_in_chunks:1 + 2 * n_in_chunks]
        w1, b1, w2, b2 = refs[1 + 2 * n_in_chunks:5 + 2 * n_in_chunks]
        rest = refs[5 + 2 * n_in_chunks:]

        scale = scale_ref[0]
        h = jnp.concatenate([r[...] for r in h_refs], axis=1)
        a = jnp.concatenate([r[...] for r in a_refs], axis=1)
        z = scale * h + a
        z = jnp.dot(z, w1[...], preferred_element_type=jnp.float32) + b1[...]
        z = jnp.maximum(z, 0.0)
        z = jnp.dot(z, w2[...], preferred_element_type=jnp.float32) + b2[...]
        z = jnp.maximum(z, 0.0)
        if final:
            wo, bo, outp = rest
            outp[...] = (jnp.dot(z, wo[...], preferred_element_type=jnp.float32)
                         + bo[...])
        else:
            outs = rest
            for c in range(n_out_chunks):
                outs[c][...] = z[:, c * CW:(c + 1) * CW]

    in_specs = (
        [pl.BlockSpec(memory_space=pltpu.SMEM)]
        + [pl.BlockSpec((_BR, CW), lambda i: (i, 0))] * (2 * n_in_chunks)
        + [pl.BlockSpec((d_in, HIDDEN), lambda i: (0, 0)),
           pl.BlockSpec((1, HIDDEN), lambda i: (0, 0)),
           pl.BlockSpec((HIDDEN, HIDDEN), lambda i: (0, 0)),
           pl.BlockSpec((1, HIDDEN), lambda i: (0, 0))]
    )
    if final:
        in_specs += [pl.BlockSpec((HIDDEN, CW), lambda i: (0, 0)),
                     pl.BlockSpec((1, CW), lambda i: (0, 0))]
        out_specs = pl.BlockSpec((_BR, CW), lambda i: (i, 0))
        out_shape = jax.ShapeDtypeStruct((N_NODES, CW), jnp.float32)
    else:
        out_specs = [pl.BlockSpec((_BR, CW), lambda i: (i, 0))] * n_out_chunks
        out_shape = [jax.ShapeDtypeStruct((N_NODES, CW), jnp.float32)] * n_out_chunks

    return pl.pallas_call(body, grid=(N_NODES // _BR,), in_specs=in_specs,
                          out_specs=out_specs, out_shape=out_shape)


def kernel(x, edge_index, params):
    src = edge_index[0].astype(jnp.int32)
    dst = edge_index[1].astype(jnp.int32)
    pad = E_PAD - N_EDGES
    src_p = jnp.concatenate([src, jnp.zeros((pad,), jnp.int32)])
    dst_pad = N_NODES + (jnp.arange(pad, dtype=jnp.int32) % (ACC_ROWS - N_NODES))
    dst_p = jnp.concatenate([dst, dst_pad])
    src_r = src_p.reshape(NT, NB, EDGE_BATCH)
    dst_r = dst_p.reshape(NT, NB, EDGE_BATCH)

    h_chunks = [x[:, c * CW:(c + 1) * CW] for c in range(D_FEAT // CW)]
    layers = params["layers"]
    for i, layer in enumerate(layers):
        n_in = len(h_chunks)
        agg_chunks = _sc_agg(n_in)(*h_chunks, src_r, dst_r)
        scale = (1.0 + layer["eps"]).reshape(1)
        b1 = layer["b1"].reshape(1, HIDDEN)
        b2 = layer["b2"].reshape(1, HIDDEN)
        if i < len(layers) - 1:
            h_chunks = _mlp(n_in, False)(
                scale, *h_chunks, *agg_chunks, layer["w1"], b1, layer["w2"], b2)
        else:
            wo = jnp.zeros((HIDDEN, CW), jnp.float32).at[:, :2].set(params["wout"])
            bo = jnp.zeros((1, CW), jnp.float32).at[0, :2].set(params["bout"])
            outp = _mlp(n_in, True)(
                scale, *h_chunks, *agg_chunks, layer["w1"], b1, layer["w2"], b2,
                wo, bo)
    return outp[:, :2]


# speedup vs baseline: 3.3475x; 1.2182x over previous
import functools

import jax
import jax.numpy as jnp
from jax import lax
from jax.experimental import pallas as pl
from jax.experimental.pallas import tpu as pltpu
from jax.experimental.pallas import tpu_sc as plsc

N_NODES = 10000
N_EDGES = 160000
D_FEAT = 256
HIDDEN = 512

NC = 2
NT = 16
CW = 128

ACC_ROWS = 10240
ROWS_PER_TILE = ACC_ROWS // NT
LAST_ROWS = N_NODES - (NT - 1) * ROWS_PER_TILE
EDGE_BATCH = 128
E_PAD = 163840
NB = E_PAD // (NT * EDGE_BATCH)


@functools.lru_cache(maxsize=None)
def _sc_agg(n_chunks):
    cps = n_chunks // NC
    mesh = plsc.VectorSubcoreMesh(
        core_axis_name="c", subcore_axis_name="s", num_cores=NC, num_subcores=NT
    )
    out_type = [jax.ShapeDtypeStruct((N_NODES, CW), jnp.float32)
                for _ in range(n_chunks)]
    scratch = [
        pltpu.VMEM((NB // 2, EDGE_BATCH), jnp.int32),
        pltpu.VMEM((NB // 2, EDGE_BATCH), jnp.int32),
        pltpu.VMEM((EDGE_BATCH, CW), jnp.float32),
        pltpu.VMEM((EDGE_BATCH, CW), jnp.float32),
        pltpu.VMEM((16, CW), jnp.float32),
        pltpu.VMEM_SHARED((ACC_ROWS, CW), jnp.float32),
        pltpu.SemaphoreType.DMA,
        pltpu.SemaphoreType.DMA,
    ]

    @functools.partial(pl.kernel, out_type=out_type, mesh=mesh,
                       scratch_types=scratch)
    def agg_kernel(*refs):
        h_refs = refs[:n_chunks]
        src_hbm = refs[n_chunks]
        dst_hbm = refs[n_chunks + 1]
        out_refs = refs[n_chunks + 2: 2 * n_chunks + 2]
        sidx, didx, gbuf0, gbuf1, zbuf, acc, gsem0, gsem1 = \
            refs[2 * n_chunks + 2:]

        cid = lax.axis_index("c")
        sid = lax.axis_index("s")
        base = sid * ROWS_PER_TILE
        nh = NB // 2

        for i in range(16):
            for j in range(CW // 16):
                zbuf[i, pl.ds(j * 16, 16)] = jnp.zeros((16,), jnp.float32)

        def process_chunk(h_ref, out_ref):
            def zero_body(k, carry):
                pltpu.sync_copy(zbuf, acc.at[pl.ds(base + k * 16, 16)])
                return carry
            lax.fori_loop(0, ROWS_PER_TILE // 16, zero_body, 0)
            plsc.subcore_barrier()

            def start_gather(b, gbuf, gsem):
                pltpu.async_copy(h_ref.at[sidx.at[b]], gbuf, gsem)

            def drain_gather(gbuf, gsem):
                pltpu.make_async_copy(h_ref.at[sidx.at[0]], gbuf, gsem).wait()

            def sub_iter(b, gbuf, gsem):
                drain_gather(gbuf, gsem)
                pltpu.sync_copy(gbuf, acc.at[didx.at[b]], add=True)

                @pl.when(b + 2 < nh)
                def _():
                    start_gather(b + 2, gbuf, gsem)

            for half in range(2):
                pltpu.sync_copy(src_hbm.at[sid, pl.ds(half * nh, nh)], sidx)
                pltpu.sync_copy(dst_hbm.at[sid, pl.ds(half * nh, nh)], didx)
                start_gather(0, gbuf0, gsem0)
                start_gather(1, gbuf1, gsem1)

                def edge_body(g, carry):
                    sub_iter(2 * g, gbuf0, gsem0)
                    sub_iter(2 * g + 1, gbuf1, gsem1)
                    return carry
                lax.fori_loop(0, nh // 2, edge_body, 0)
            plsc.subcore_barrier()

            @pl.when(sid < NT - 1)
            def _():
                pltpu.sync_copy(acc.at[pl.ds(base, ROWS_PER_TILE)],
                                out_ref.at[pl.ds(base, ROWS_PER_TILE)])

            @pl.when(sid == NT - 1)
            def _():
                pltpu.sync_copy(
                    acc.at[pl.ds((NT - 1) * ROWS_PER_TILE, LAST_ROWS)],
                    out_ref.at[pl.ds((NT - 1) * ROWS_PER_TILE, LAST_ROWS)])
            plsc.subcore_barrier()

        for j in range(cps):
            for side in range(NC):
                chunk = side * cps + j

                @pl.when(cid == side)
                def _(chunk=chunk):
                    process_chunk(h_refs[chunk], out_refs[chunk])

    return agg_kernel


_BR = 2000


@functools.lru_cache(maxsize=None)
def _mlp(n_in_chunks, final):
    d_in = n_in_chunks * CW
    n_out_chunks = HIDDEN // CW

    def body(*refs):
        scale_ref = refs[0]
        h_refs = refs[1:1 + n_in_chunks]
        a_refs = refs[1 + n_in_chunks:1 + 2 * n_in_chunks]
        w1, b1, w2, b2 = refs[1 + 2 * n_in_chunks:5 + 2 * n_in_chunks]
        rest = refs[5 + 2 * n_in_chunks:]

        scale = scale_ref[0]
        h = jnp.concatenate([r[...] for r in h_refs], axis=1)
        a = jnp.concatenate([r[...] for r in a_refs], axis=1)
        z = scale * h + a
        z = jnp.dot(z, w1[...], preferred_element_type=jnp.float32) + b1[...]
        z = jnp.maximum(z, 0.0)
        z = jnp.dot(z, w2[...], preferred_element_type=jnp.float32) + b2[...]
        z = jnp.maximum(z, 0.0)
        if final:
            wo, bo, outp = rest
            outp[...] = (jnp.dot(z, wo[...], preferred_element_type=jnp.float32)
                         + bo[...])
        else:
            outs = rest
            for c in range(n_out_chunks):
                outs[c][...] = z[:, c * CW:(c + 1) * CW]

    in_specs = (
        [pl.BlockSpec(memory_space=pltpu.SMEM)]
        + [pl.BlockSpec((_BR, CW), lambda i: (i, 0))] * (2 * n_in_chunks)
        + [pl.BlockSpec((d_in, HIDDEN), lambda i: (0, 0)),
           pl.BlockSpec((1, HIDDEN), lambda i: (0, 0)),
           pl.BlockSpec((HIDDEN, HIDDEN), lambda i: (0, 0)),
           pl.BlockSpec((1, HIDDEN), lambda i: (0, 0))]
    )
    if final:
        in_specs += [pl.BlockSpec((HIDDEN, CW), lambda i: (0, 0)),
                     pl.BlockSpec((1, CW), lambda i: (0, 0))]
        out_specs = pl.BlockSpec((_BR, CW), lambda i: (i, 0))
        out_shape = jax.ShapeDtypeStruct((N_NODES, CW), jnp.float32)
    else:
        out_specs = [pl.BlockSpec((_BR, CW), lambda i: (i, 0))] * n_out_chunks
        out_shape = [jax.ShapeDtypeStruct((N_NODES, CW), jnp.float32)] * n_out_chunks

    return pl.pallas_call(body, grid=(N_NODES // _BR,), in_specs=in_specs,
                          out_specs=out_specs, out_shape=out_shape)


def kernel(x, edge_index, params):
    src = edge_index[0].astype(jnp.int32)
    dst = edge_index[1].astype(jnp.int32)
    pad = E_PAD - N_EDGES
    src_p = jnp.concatenate([src, jnp.zeros((pad,), jnp.int32)])
    dst_pad = N_NODES + (jnp.arange(pad, dtype=jnp.int32) % (ACC_ROWS - N_NODES))
    dst_p = jnp.concatenate([dst, dst_pad])
    src_r = src_p.reshape(NT, NB, EDGE_BATCH)
    dst_r = dst_p.reshape(NT, NB, EDGE_BATCH)

    h_chunks = [x[:, c * CW:(c + 1) * CW] for c in range(D_FEAT // CW)]
    layers = params["layers"]
    for i, layer in enumerate(layers):
        n_in = len(h_chunks)
        agg_chunks = _sc_agg(n_in)(*h_chunks, src_r, dst_r)
        scale = (1.0 + layer["eps"]).reshape(1)
        b1 = layer["b1"].reshape(1, HIDDEN)
        b2 = layer["b2"].reshape(1, HIDDEN)
        if i < len(layers) - 1:
            h_chunks = _mlp(n_in, False)(
                scale, *h_chunks, *agg_chunks, layer["w1"], b1, layer["w2"], b2)
        else:
            wo = jnp.zeros((HIDDEN, CW), jnp.float32).at[:, :2].set(params["wout"])
            bo = jnp.zeros((1, CW), jnp.float32).at[0, :2].set(params["bout"])
            outp = _mlp(n_in, True)(
                scale, *h_chunks, *agg_chunks, layer["w1"], b1, layer["w2"], b2,
                wo, bo)
    return outp[:, :2]
